# baseline (speedup 1.0000x reference)
import functools

import jax
import jax.numpy as jnp
from jax import lax
from jax.experimental import pallas as pl
from jax.experimental.pallas import tpu as pltpu
from jax.experimental.pallas import tpu_sc as plsc

F32 = jnp.float32
I32 = jnp.int32

_N = 10000
_E = 160000
_K = 2
_NC = 2
_NS = 16
_NW = _NC * _NS
_EB = 64
_BPW = 80
_NBLK = _NW * _BPW
_NBLK_P = 2624
_EPAD = _NBLK_P * _EB
_CH = 64
_NPAD = 10240
_RPT = _NPAD // _NS

_mesh = plsc.VectorSubcoreMesh(
    core_axis_name="c", subcore_axis_name="s", num_cores=_NC, num_subcores=_NS)
_sc_params = pltpu.CompilerParams(use_tc_tiling_on_sc=False)


_GWB = 2048


def _gw_body(attr_ref, mus_ref, iss_ref, out_ref):
    i = pl.program_id(0)
    eidx = i * _GWB + lax.broadcasted_iota(I32, (1, _GWB), 1)
    valid = (eidx < _E).astype(F32)
    rows = []
    for l in range(4):
        for k in range(_K):
            q = jnp.zeros((1, _GWB), F32)
            for p in range(3):
                d = (attr_ref[p:p + 1, :] - mus_ref[l, k, p]) * iss_ref[l, k, p]
                q = q + d * d
            rows.append(jnp.exp(-0.5 * q) * valid)
    rows.append(valid)
    for _ in range(7):
        rows.append(jnp.zeros((1, _GWB), F32))
    out_ref[...] = jnp.concatenate(rows, axis=0)


def _gw_call(attr_t, mus, iss):
    return pl.pallas_call(
        _gw_body,
        grid=(_EPAD // _GWB,),
        in_specs=[
            pl.BlockSpec((3, _GWB), lambda i: (0, i)),
            pl.BlockSpec(memory_space=pltpu.SMEM),
            pl.BlockSpec(memory_space=pltpu.SMEM),
        ],
        out_specs=pl.BlockSpec((16, _GWB), lambda i: (0, i)),
        out_shape=jax.ShapeDtypeStruct((16, _EPAD), F32),
        compiler_params=pltpu.CompilerParams(
            dimension_semantics=("parallel",)),
    )(attr_t, mus, iss)


def _edge_pipeline(nb, table, srcv, dstv, rows2, msg2, acc,
                   gs0, gs1, ss0, ss1, compute_block):
    gsems = (gs0, gs1)
    ssems = (ss0, ss1)

    def g_start(b, par):
        pltpu.make_async_copy(
            table.at[srcv.at[b]], rows2.at[par], gsems[par]).start()

    def g_wait(b, par):
        pltpu.make_async_copy(
            table.at[srcv.at[b]], rows2.at[par], gsems[par]).wait()

    def s_start(b, par):
        pltpu.async_copy(
            msg2.at[par], acc.at[dstv.at[b]], ssems[par], add=True)

    def s_wait(b, par):
        pltpu.make_async_copy(
            msg2.at[par], acc.at[dstv.at[b]], ssems[par]).wait()

    g_start(0, 0)
    g_start(1, 1)

    @pl.loop(0, nb // 2)
    def _pair(bb):
        for par in (0, 1):
            b = 2 * bb + par
            g_wait(b, par)

            @pl.when(b >= 2)
            def _():
                s_wait(b - 2, par)

            compute_block(b, par)

            @pl.when(b + 2 < nb)
            def _():
                g_start(b + 2, par)

            s_start(b, par)

    s_wait(nb - 2, 0)
    s_wait(nb - 1, 1)


def _sc1_body(B0, B1, BMAX, feat_hbm, src_hbm, dst_hbm, gw_hbm, zeros_hbm,
              out_hbm, srcv, dstv, g0v, g1v, wv, rows2, msg2, acc,
              gs0, gs1, ss0, ss1):
    cid = lax.axis_index("c")
    sid = lax.axis_index("s")
    nb = jnp.where(cid == 0, B0, B1)
    base = jnp.where(cid == 0, sid * B0, _NS * B0 + sid * B1)
    lane = lax.broadcasted_iota(I32, (16,), 0)

    pltpu.sync_copy(zeros_hbm.at[:, pl.ds(0, 32)],
                    acc.at[pl.ds(sid * _RPT, _RPT)])
    pltpu.sync_copy(src_hbm.at[pl.ds(base, BMAX)], srcv)
    pltpu.sync_copy(dst_hbm.at[pl.ds(base, BMAX)], dstv)
    pltpu.sync_copy(gw_hbm.at[0, pl.ds(base, BMAX)], g0v)
    pltpu.sync_copy(gw_hbm.at[1, pl.ds(base, BMAX)], g1v)
    pltpu.sync_copy(gw_hbm.at[8, pl.ds(base, BMAX)], wv)
    plsc.subcore_barrier()

    def compute_block(b, par):
        @pl.loop(0, _EB // 16)
        def _grp(g):
            g0vec = g0v[b, pl.ds(16 * g, 16)]
            g1vec = g1v[b, pl.ds(16 * g, 16)]
            wvec = wv[b, pl.ds(16 * g, 16)]
            for j in range(16):
                e = 16 * g + j
                v = rows2[par, e, :]
                m0 = jnp.where(lane < 3, v * g0vec[j], 0.0)
                m1 = jnp.where(lane < 3, v * g1vec[j],
                               jnp.where(lane == 3, wvec[j], 0.0))
                msg2[par, e, pl.ds(0, 16)] = m0
                msg2[par, e, pl.ds(16, 16)] = m1

    _edge_pipeline(nb, feat_hbm, srcv, dstv, rows2, msg2, acc,
                   gs0, gs1, ss0, ss1, compute_block)

    plsc.subcore_barrier()
    pltpu.sync_copy(acc.at[pl.ds(sid * _RPT, _RPT)],
                    out_hbm.at[cid, pl.ds(sid * _RPT, _RPT)])


_B0_1, _B1_1 = 84, 76
_sc1_call = functools.partial(
    pl.kernel,
    out_type=jax.ShapeDtypeStruct((_NC, _NPAD, 32), F32),
    mesh=_mesh,
    compiler_params=_sc_params,
    scratch_types=[
        pltpu.VMEM((_B0_1, _EB), I32),
        pltpu.VMEM((_B0_1, _EB), I32),
        pltpu.VMEM((_B0_1, _EB), F32),
        pltpu.VMEM((_B0_1, _EB), F32),
        pltpu.VMEM((_B0_1, _EB), F32),
        pltpu.VMEM((2, _EB, 16), F32),
        pltpu.VMEM((2, _EB, 32), F32),
        pltpu.VMEM_SHARED((_NPAD, 32), F32),
        pltpu.SemaphoreType.DMA,
        pltpu.SemaphoreType.DMA,
        pltpu.SemaphoreType.DMA,
        pltpu.SemaphoreType.DMA,
    ],
)(functools.partial(_sc1_body, _B0_1, _B1_1, _B0_1))


def _scl_body(layer, nc, B0, B1, BMAX, *refs):
    xcs = refs[:nc]
    (src_hbm, dst_hbm, gw_hbm, zeros_hbm, out_hbm,
     srcv, dstv, g0v, g1v, rows2, msg2, acc, gs0, gs1, ss0, ss1) = refs[nc:]
    cid = lax.axis_index("c")
    sid = lax.axis_index("s")
    nb = jnp.where(cid == 0, B0, B1)
    base = jnp.where(cid == 0, sid * B0, _NS * B0 + sid * B1)

    pltpu.sync_copy(src_hbm.at[pl.ds(base, BMAX)], srcv)
    pltpu.sync_copy(dst_hbm.at[pl.ds(base, BMAX)], dstv)
    pltpu.sync_copy(gw_hbm.at[2 * layer, pl.ds(base, BMAX)], g0v)
    pltpu.sync_copy(gw_hbm.at[2 * layer + 1, pl.ds(base, BMAX)], g1v)

    def compute_block(b, par):
        @pl.loop(0, _EB // 16)
        def _grp(g):
            g0vec = g0v[b, pl.ds(16 * g, 16)]
            g1vec = g1v[b, pl.ds(16 * g, 16)]
            for jj in range(16):
                e = 16 * g + jj
                g0 = g0vec[jj]
                g1 = g1vec[jj]
                for j in range(_CH // 16):
                    v = rows2[par, e, pl.ds(16 * j, 16)]
                    msg2[par, e, pl.ds(16 * j, 16)] = v * g0
                    msg2[par, e, pl.ds(_CH + 16 * j, 16)] = v * g1

    for ci in range(nc):
        pltpu.sync_copy(zeros_hbm, acc.at[pl.ds(sid * _RPT, _RPT)])
        plsc.subcore_barrier()
        _edge_pipeline(nb, xcs[ci], srcv, dstv, rows2, msg2, acc,
                       gs0, gs1, ss0, ss1, compute_block)
        plsc.subcore_barrier()
        pltpu.sync_copy(acc.at[pl.ds(sid * _RPT, _RPT)],
                        out_hbm.at[cid, ci, pl.ds(sid * _RPT, _RPT)])


_B0_23, _B1_23 = 94, 66


def _scl_call(layer, nc):
    return functools.partial(
        pl.kernel,
        out_type=jax.ShapeDtypeStruct((_NC, nc, _NPAD, 2 * _CH), F32),
        mesh=_mesh,
        compiler_params=_sc_params,
        scratch_types=[
            pltpu.VMEM((_B0_23, _EB), I32),
            pltpu.VMEM((_B0_23, _EB), I32),
            pltpu.VMEM((_B0_23, _EB), F32),
            pltpu.VMEM((_B0_23, _EB), F32),
            pltpu.VMEM((2, _EB, _CH), F32),
            pltpu.VMEM((2, _EB, 2 * _CH), F32),
            pltpu.VMEM_SHARED((_NPAD, 2 * _CH), F32),
            pltpu.SemaphoreType.DMA,
            pltpu.SemaphoreType.DMA,
            pltpu.SemaphoreType.DMA,
            pltpu.SemaphoreType.DMA,
        ],
    )(functools.partial(_scl_body, layer, nc, _B0_23, _B1_23, _B0_23))


_EB4 = 128


def _sc4_body(B0, B1, BMAX, h0, h1, h2, h3, src_hbm, dst_hbm, gw_hbm,
              zeros_hbm, ptab_hbm, out_hbm, srcv, dstv, g0v, g1v, rows2,
              msg2, ptab, acc, gs0, gs1, ss0, ss1):
    hts = (h0, h1, h2, h3)
    cid = lax.axis_index("c")
    sid = lax.axis_index("s")
    nb = jnp.where(cid == 0, B0, B1)
    base = jnp.where(cid == 0, sid * B0, _NS * B0 + sid * B1)

    del ptab_hbm, ptab
    pltpu.sync_copy(src_hbm.at[pl.ds(base, BMAX)], srcv)
    pltpu.sync_copy(dst_hbm.at[pl.ds(base, BMAX)], dstv)
    pltpu.sync_copy(gw_hbm.at[6, pl.ds(base, BMAX)], g0v)
    pltpu.sync_copy(gw_hbm.at[7, pl.ds(base, BMAX)], g1v)

    def compute_block(b, par):
        @pl.loop(0, _EB4 // 16)
        def _grp(g):
            g0vec = g0v[b, pl.ds(16 * g, 16)]
            g1vec = g1v[b, pl.ds(16 * g, 16)]
            for jj in range(16):
                e = 16 * g + jj
                g0 = g0vec[jj]
                g1 = g1vec[jj]
                hs = []
                for t in range(4):
                    w = rows2[par, e, pl.ds(16 * t, 16)]
                    for u in (jnp.bitwise_and(w, 65535),
                              jnp.bitwise_and(
                                  jax.lax.shift_right_logical(w, 16), 65535)):
                        q = jnp.bitwise_and(
                            jax.lax.shift_right_logical(u, 7), 255)
                        r = jnp.bitwise_and(u, 127)
                        p2 = jnp.exp(q.astype(F32) * 0.6931471805599453
                                     - 92.88172219703817)
                        va = (r + 128).astype(F32) * p2
                        hs.append(jnp.where(u >= 32768, -va, va))
                for j in range(_CH // 16):
                    msg2[par, e, pl.ds(16 * j, 16)] = (
                        hs[j] * g0 + hs[4 + j] * g1)

    for ci in range(4):
        pltpu.sync_copy(zeros_hbm.at[:, pl.ds(0, _CH)],
                        acc.at[pl.ds(sid * _RPT, _RPT)])
        plsc.subcore_barrier()
        _edge_pipeline(nb, hts[ci], srcv, dstv, rows2, msg2, acc,
                       gs0, gs1, ss0, ss1, compute_block)
        plsc.subcore_barrier()
        pltpu.sync_copy(acc.at[pl.ds(sid * _RPT, _RPT)],
                        out_hbm.at[cid, ci, pl.ds(sid * _RPT, _RPT)])


_B0_4, _B1_4 = 54, 26
_sc4_call = functools.partial(
    pl.kernel,
    out_type=jax.ShapeDtypeStruct((_NC, 4, _NPAD, _CH), F32),
    mesh=_mesh,
    compiler_params=_sc_params,
    scratch_types=[
        pltpu.VMEM((_B0_4, _EB4), I32),
        pltpu.VMEM((_B0_4, _EB4), I32),
        pltpu.VMEM((_B0_4, _EB4), F32),
        pltpu.VMEM((_B0_4, _EB4), F32),
        pltpu.VMEM((2, _EB4, _CH), I32),
        pltpu.VMEM((2, _EB4, _CH), F32),
        pltpu.VMEM((256,), F32),
        pltpu.VMEM_SHARED((_NPAD, _CH), F32),
        pltpu.SemaphoreType.DMA,
        pltpu.SemaphoreType.DMA,
        pltpu.SemaphoreType.DMA,
        pltpu.SemaphoreType.DMA,
    ],
)(functools.partial(_sc4_body, _B0_4, _B1_4, _B0_4))


_BN = 1024


def _tc1_body(a_ref, w_ref, b_ref, x_ref, invd_ref):
    a = a_ref[0] + a_ref[1]
    a6 = jnp.concatenate([a[:, 0:3], a[:, 16:19]], axis=1)
    s = jnp.dot(a6, w_ref[...], preferred_element_type=F32)
    deg = a[:, 19:20]
    inv = 1.0 / jnp.maximum(deg, 1.0)
    x_ref[...] = s * inv + b_ref[...]
    invd_ref[...] = jnp.broadcast_to(inv, (_BN, 8))


def _tc1_call(a1, wstk, bias):
    return pl.pallas_call(
        _tc1_body,
        grid=(_NPAD // _BN,),
        in_specs=[
            pl.BlockSpec((_NC, _BN, 32), lambda i: (0, i, 0)),
            pl.BlockSpec((6, 64), lambda i: (0, 0)),
            pl.BlockSpec((1, 64), lambda i: (0, 0)),
        ],
        out_specs=[
            pl.BlockSpec((_BN, 64), lambda i: (i, 0)),
            pl.BlockSpec((_BN, 8), lambda i: (i, 0)),
        ],
        out_shape=[
            jax.ShapeDtypeStruct((_N, 64), F32),
            jax.ShapeDtypeStruct((_N, 8), F32),
        ],
        compiler_params=pltpu.CompilerParams(
            dimension_semantics=("parallel",)),
    )(a1, wstk, bias)


def _tc2_body(a_ref, w_ref, invd_ref, b_ref, o0_ref, o1_ref):
    a = a_ref[0, 0] + a_ref[1, 0]
    s = jnp.dot(a, w_ref[0], preferred_element_type=F32)
    res = s * invd_ref[:, 0:1] + b_ref[...]
    o0_ref[...] = res[:, 0:64]
    o1_ref[...] = res[:, 64:128]


def _tc2_call(a, wstk, invd, bias):
    return pl.pallas_call(
        _tc2_body,
        grid=(_NPAD // _BN,),
        in_specs=[
            pl.BlockSpec((_NC, 1, _BN, 128), lambda i: (0, 0, i, 0)),
            pl.BlockSpec((1, 128, 128), lambda i: (0, 0, 0)),
            pl.BlockSpec((_BN, 8), lambda i: (i, 0)),
            pl.BlockSpec((1, 128), lambda i: (0, 0)),
        ],
        out_specs=[pl.BlockSpec((_BN, 64), lambda i: (i, 0))
                   for _ in range(2)],
        out_shape=[jax.ShapeDtypeStruct((_N, 64), F32) for _ in range(2)],
        compiler_params=pltpu.CompilerParams(
            dimension_semantics=("parallel",)),
    )(a, wstk, invd, bias)


def _tc3_body(a_ref, w_ref, invd_ref, b_ref, w4_ref, *rest):
    ci = pl.program_id(1)
    out_refs = rest[:4]
    accr = rest[4]
    a = a_ref[0, 0] + a_ref[1, 0]
    p = jnp.dot(a, w_ref[0], preferred_element_type=F32)

    @pl.when(ci == 0)
    def _():
        accr[...] = p

    @pl.when(ci > 0)
    def _():
        accr[...] += p

    @pl.when(ci == 1)
    def _():
        x4 = accr[...] * invd_ref[:, 0:1] + b_ref[...]
        h = jnp.dot(x4, w4_ref[...], preferred_element_type=F32)
        for co in range(4):
            out_refs[co][...] = h[:, 128 * co:128 * (co + 1)].astype(
                jnp.bfloat16)


def _tc3_call(a, wstk, invd, bias, w4p):
    return pl.pallas_call(
        _tc3_body,
        grid=(_NPAD // _BN, 2),
        in_specs=[
            pl.BlockSpec((_NC, 1, _BN, 128), lambda i, ci: (0, ci, i, 0)),
            pl.BlockSpec((1, 128, 256), lambda i, ci: (ci, 0, 0)),
            pl.BlockSpec((_BN, 8), lambda i, ci: (i, 0)),
            pl.BlockSpec((1, 256), lambda i, ci: (0, 0)),
            pl.BlockSpec((256, 512), lambda i, ci: (0, 0)),
        ],
        out_specs=[pl.BlockSpec((_BN, 128), lambda i, ci: (i, 0))
                   for _ in range(4)],
        out_shape=[jax.ShapeDtypeStruct((_N, 128), jnp.bfloat16)
                   for _ in range(4)],
        scratch_shapes=[pltpu.VMEM((_BN, 256), F32)],
        compiler_params=pltpu.CompilerParams(
            dimension_semantics=("parallel", "arbitrary")),
    )(a, wstk, invd, bias, w4p)


def _tc4_body(a_ref, invd_ref, b_ref, out_ref):
    inv = invd_ref[:, 0:1]
    parts = [(a_ref[0, co] + a_ref[1, co]) * inv for co in range(4)]
    out_ref[...] = jnp.concatenate(parts, axis=1) + b_ref[...]


def _tc4_call(a, invd, bias):
    return pl.pallas_call(
        _tc4_body,
        grid=(_NPAD // _BN,),
        in_specs=[
            pl.BlockSpec((_NC, 4, _BN, 64), lambda i: (0, 0, i, 0)),
            pl.BlockSpec((_BN, 8), lambda i: (i, 0)),
            pl.BlockSpec((1, 256), lambda i: (0, 0)),
        ],
        out_specs=pl.BlockSpec((_BN, 256), lambda i: (i, 0)),
        out_shape=jax.ShapeDtypeStruct((_N, 256), F32),
        compiler_params=pltpu.CompilerParams(
            dimension_semantics=("parallel",)),
    )(a, invd, bias)


def _stack_w(W, out_d):
    in_d = W.shape[0]
    nc = in_d // _CH
    w0 = W[:, :out_d].reshape(nc, _CH, out_d)
    w1 = W[:, out_d:].reshape(nc, _CH, out_d)
    return jnp.concatenate([w0, w1], axis=1)


def kernel(features, edge_index, edge_attr, W1, mu1, is1, b1, W2, mu2, is2,
           b2, W3, mu3, is3, b3, W4, mu4, is4, b4):
    src2d = jnp.pad(edge_index[0], (0, _EPAD - _E)).reshape(_NBLK_P, _EB)
    dst2d = jnp.pad(edge_index[1], (0, _EPAD - _E)).reshape(_NBLK_P, _EB)
    attr_t = jnp.pad(edge_attr, ((0, _EPAD - _E), (0, 0))).T
    feat16 = jnp.pad(features, ((0, 0), (0, 13)))
    mus = jnp.stack([mu1, mu2, mu3, mu4])
    iss = jnp.stack([is1, is2, is3, is4])
    zeros = jnp.zeros((_RPT, 2 * _CH), F32)

    gw = _gw_call(attr_t, mus, iss).reshape(16, _NBLK_P, _EB)

    a1 = _sc1_call(feat16, src2d, dst2d, gw, zeros)
    wstk1 = jnp.concatenate([W1[:, :64], W1[:, 64:]], axis=0)
    x2, invd = _tc1_call(a1, wstk1, b1.reshape(1, 64))

    a2 = _scl_call(1, 1)(x2, src2d, dst2d, gw, zeros)
    x3 = _tc2_call(a2, _stack_w(W2, 128), invd, b2.reshape(1, 128))

    a3 = _scl_call(2, 2)(*x3, src2d, dst2d, gw, zeros)
    w4p = jnp.concatenate(
        [jnp.concatenate([W4[:, 64 * co:64 * (co + 1)],
                          W4[:, 256 + 64 * co:256 + 64 * (co + 1)]], axis=1)
         for co in range(4)], axis=1)
    w4p = w4p.reshape(256, 16, 2, 16).transpose(0, 1, 3, 2).reshape(256, 512)
    h4 = _tc3_call(a3, _stack_w(W3, 256), invd, b3.reshape(1, 256), w4p)

    src2d4 = src2d.reshape(_NBLK_P // 2, _EB4)
    dst2d4 = dst2d.reshape(_NBLK_P // 2, _EB4)
    gw4 = gw.reshape(16, _NBLK_P // 2, _EB4)
    h4i = [lax.bitcast_convert_type(h.reshape(_N, 64, 2), I32) for h in h4]
    ptab = (2.0 ** (jnp.arange(256, dtype=F32) - 134.0)).astype(F32)
    a4 = _sc4_call(*h4i, src2d4, dst2d4, gw4, zeros, ptab)
    return _tc4_call(a4, invd, b4.reshape(1, 256))


# speedup vs baseline: 1.4497x; 1.4497x over previous
import functools

import jax
import jax.numpy as jnp
from jax import lax
from jax.experimental import pallas as pl
from jax.experimental.pallas import tpu as pltpu
from jax.experimental.pallas import tpu_sc as plsc

F32 = jnp.float32
I32 = jnp.int32

_N = 10000
_E = 160000
_K = 2
_NC = 2
_NS = 16
_NW = _NC * _NS
_EB = 64
_BPW = 80
_NBLK = _NW * _BPW
_NBLK_P = 2624
_EPAD = _NBLK_P * _EB
_CH = 64
_NPAD = 10240
_RPT = _NPAD // _NS

_mesh = plsc.VectorSubcoreMesh(
    core_axis_name="c", subcore_axis_name="s", num_cores=_NC, num_subcores=_NS)
_sc_params = pltpu.CompilerParams(use_tc_tiling_on_sc=False)


_GWB = 2048


def _gw_body(attr_ref, mus_ref, iss_ref, out_ref):
    i = pl.program_id(0)
    eidx = i * _GWB + lax.broadcasted_iota(I32, (1, _GWB), 1)
    valid = (eidx < _E).astype(F32)
    rows = []
    for l in range(4):
        for k in range(_K):
            q = jnp.zeros((1, _GWB), F32)
            for p in range(3):
                d = (attr_ref[p:p + 1, :] - mus_ref[l, k, p]) * iss_ref[l, k, p]
                q = q + d * d
            rows.append(jnp.exp(-0.5 * q) * valid)
    rows.append(valid)
    for _ in range(7):
        rows.append(jnp.zeros((1, _GWB), F32))
    out_ref[...] = jnp.concatenate(rows, axis=0)


def _gw_call(attr_t, mus, iss):
    return pl.pallas_call(
        _gw_body,
        grid=(_EPAD // _GWB,),
        in_specs=[
            pl.BlockSpec((3, _GWB), lambda i: (0, i)),
            pl.BlockSpec(memory_space=pltpu.SMEM),
            pl.BlockSpec(memory_space=pltpu.SMEM),
        ],
        out_specs=pl.BlockSpec((16, _GWB), lambda i: (0, i)),
        out_shape=jax.ShapeDtypeStruct((16, _EPAD), F32),
        compiler_params=pltpu.CompilerParams(
            dimension_semantics=("parallel",)),
    )(attr_t, mus, iss)


def _edge_pipeline(nb, table, srcv, dstv, rows2, msg2, acc,
                   gs0, gs1, ss0, ss1, compute_block):
    gsems = (gs0, gs1)
    ssems = (ss0, ss1)

    def g_start(b, par):
        pltpu.make_async_copy(
            table.at[srcv.at[b]], rows2.at[par], gsems[par]).start()

    def g_wait(b, par):
        pltpu.make_async_copy(
            table.at[srcv.at[b]], rows2.at[par], gsems[par]).wait()

    def s_start(b, par):
        pltpu.async_copy(
            msg2.at[par], acc.at[dstv.at[b]], ssems[par], add=True)

    def s_wait(b, par):
        pltpu.make_async_copy(
            msg2.at[par], acc.at[dstv.at[b]], ssems[par]).wait()

    g_start(0, 0)
    g_start(1, 1)

    @pl.loop(0, nb // 2)
    def _pair(bb):
        for par in (0, 1):
            b = 2 * bb + par
            g_wait(b, par)

            @pl.when(b >= 2)
            def _():
                s_wait(b - 2, par)

            compute_block(b, par)

            @pl.when(b + 2 < nb)
            def _():
                g_start(b + 2, par)

            s_start(b, par)

    s_wait(nb - 2, 0)
    s_wait(nb - 1, 1)


def _sc1_body(B0, B1, BMAX, feat_hbm, src_hbm, dst_hbm, gw_hbm, zeros_hbm,
              out_hbm, srcv, dstv, g0v, g1v, wv, rows2, msg2, acc,
              gs0, gs1, ss0, ss1):
    cid = lax.axis_index("c")
    sid = lax.axis_index("s")
    nb = jnp.where(cid == 0, B0, B1)
    base = jnp.where(cid == 0, sid * B0, _NS * B0 + sid * B1)
    lane = lax.broadcasted_iota(I32, (16,), 0)

    pltpu.sync_copy(zeros_hbm.at[:, pl.ds(0, 32)],
                    acc.at[pl.ds(sid * _RPT, _RPT)])
    pltpu.sync_copy(src_hbm.at[pl.ds(base, BMAX)], srcv)
    pltpu.sync_copy(dst_hbm.at[pl.ds(base, BMAX)], dstv)
    pltpu.sync_copy(gw_hbm.at[0, pl.ds(base, BMAX)], g0v)
    pltpu.sync_copy(gw_hbm.at[1, pl.ds(base, BMAX)], g1v)
    pltpu.sync_copy(gw_hbm.at[8, pl.ds(base, BMAX)], wv)
    plsc.subcore_barrier()

    def compute_block(b, par):
        @pl.loop(0, _EB // 16)
        def _grp(g):
            g0vec = g0v[b, pl.ds(16 * g, 16)]
            g1vec = g1v[b, pl.ds(16 * g, 16)]
            wvec = wv[b, pl.ds(16 * g, 16)]
            for j in range(16):
                e = 16 * g + j
                v = rows2[par, e, :]
                m0 = jnp.where(lane < 3, v * g0vec[j], 0.0)
                m1 = jnp.where(lane < 3, v * g1vec[j],
                               jnp.where(lane == 3, wvec[j], 0.0))
                msg2[par, e, pl.ds(0, 16)] = m0
                msg2[par, e, pl.ds(16, 16)] = m1

    _edge_pipeline(nb, feat_hbm, srcv, dstv, rows2, msg2, acc,
                   gs0, gs1, ss0, ss1, compute_block)

    plsc.subcore_barrier()
    pltpu.sync_copy(acc.at[pl.ds(sid * _RPT, _RPT)],
                    out_hbm.at[cid, pl.ds(sid * _RPT, _RPT)])


_B0_1, _B1_1 = 84, 76
_sc1_call = functools.partial(
    pl.kernel,
    out_type=jax.ShapeDtypeStruct((_NC, _NPAD, 32), F32),
    mesh=_mesh,
    compiler_params=_sc_params,
    scratch_types=[
        pltpu.VMEM((_B0_1, _EB), I32),
        pltpu.VMEM((_B0_1, _EB), I32),
        pltpu.VMEM((_B0_1, _EB), F32),
        pltpu.VMEM((_B0_1, _EB), F32),
        pltpu.VMEM((_B0_1, _EB), F32),
        pltpu.VMEM((2, _EB, 16), F32),
        pltpu.VMEM((2, _EB, 32), F32),
        pltpu.VMEM_SHARED((_NPAD, 32), F32),
        pltpu.SemaphoreType.DMA,
        pltpu.SemaphoreType.DMA,
        pltpu.SemaphoreType.DMA,
        pltpu.SemaphoreType.DMA,
    ],
)(functools.partial(_sc1_body, _B0_1, _B1_1, _B0_1))


def _scl_body(layer, nc, B0, B1, BMAX, *refs):
    xcs = refs[:nc]
    (src_hbm, dst_hbm, gw_hbm, zeros_hbm, out_hbm,
     srcv, dstv, g0v, g1v, rows2, msg2, acc, gs0, gs1, ss0, ss1) = refs[nc:]
    cid = lax.axis_index("c")
    sid = lax.axis_index("s")
    nb = jnp.where(cid == 0, B0, B1)
    base = jnp.where(cid == 0, sid * B0, _NS * B0 + sid * B1)

    pltpu.sync_copy(src_hbm.at[pl.ds(base, BMAX)], srcv)
    pltpu.sync_copy(dst_hbm.at[pl.ds(base, BMAX)], dstv)
    pltpu.sync_copy(gw_hbm.at[2 * layer, pl.ds(base, BMAX)], g0v)
    pltpu.sync_copy(gw_hbm.at[2 * layer + 1, pl.ds(base, BMAX)], g1v)

    def compute_block(b, par):
        @pl.loop(0, _EB // 16)
        def _grp(g):
            g0vec = g0v[b, pl.ds(16 * g, 16)]
            g1vec = g1v[b, pl.ds(16 * g, 16)]
            for jj in range(16):
                e = 16 * g + jj
                g0 = g0vec[jj]
                g1 = g1vec[jj]
                for j in range(_CH // 16):
                    v = rows2[par, e, pl.ds(16 * j, 16)]
                    msg2[par, e, pl.ds(16 * j, 16)] = v * g0
                    msg2[par, e, pl.ds(_CH + 16 * j, 16)] = v * g1

    for ci in range(nc):
        pltpu.sync_copy(zeros_hbm, acc.at[pl.ds(sid * _RPT, _RPT)])
        plsc.subcore_barrier()
        _edge_pipeline(nb, xcs[ci], srcv, dstv, rows2, msg2, acc,
                       gs0, gs1, ss0, ss1, compute_block)
        plsc.subcore_barrier()
        pltpu.sync_copy(acc.at[pl.ds(sid * _RPT, _RPT)],
                        out_hbm.at[cid, ci, pl.ds(sid * _RPT, _RPT)])


_B0_23, _B1_23 = 94, 66


def _scl_call(layer, nc):
    return functools.partial(
        pl.kernel,
        out_type=jax.ShapeDtypeStruct((_NC, nc, _NPAD, 2 * _CH), F32),
        mesh=_mesh,
        compiler_params=_sc_params,
        scratch_types=[
            pltpu.VMEM((_B0_23, _EB), I32),
            pltpu.VMEM((_B0_23, _EB), I32),
            pltpu.VMEM((_B0_23, _EB), F32),
            pltpu.VMEM((_B0_23, _EB), F32),
            pltpu.VMEM((2, _EB, _CH), F32),
            pltpu.VMEM((2, _EB, 2 * _CH), F32),
            pltpu.VMEM_SHARED((_NPAD, 2 * _CH), F32),
            pltpu.SemaphoreType.DMA,
            pltpu.SemaphoreType.DMA,
            pltpu.SemaphoreType.DMA,
            pltpu.SemaphoreType.DMA,
        ],
    )(functools.partial(_scl_body, layer, nc, _B0_23, _B1_23, _B0_23))


_EB4 = 128


def _sc4_body(B0, B1, BMAX, h0, h1, h2, h3, src_hbm, dst_hbm, gw_hbm,
              zeros_hbm, out_hbm, srcv, dstv, g0v, g1v, rows2, msg2, acc,
              gs0, gs1, ss0, ss1):
    hts = (h0, h1, h2, h3)
    cid = lax.axis_index("c")
    sid = lax.axis_index("s")
    nb = jnp.where(cid == 0, B0, B1)
    base = jnp.where(cid == 0, sid * B0, _NS * B0 + sid * B1)

    pltpu.sync_copy(src_hbm.at[pl.ds(base, BMAX)], srcv)
    pltpu.sync_copy(dst_hbm.at[pl.ds(base, BMAX)], dstv)
    pltpu.sync_copy(gw_hbm.at[6, pl.ds(base, BMAX)], g0v)
    pltpu.sync_copy(gw_hbm.at[7, pl.ds(base, BMAX)], g1v)

    def compute_block(b, par):
        @pl.loop(0, _EB4 // 16)
        def _grp(g):
            g0vec = g0v[b, pl.ds(16 * g, 16)]
            g1vec = g1v[b, pl.ds(16 * g, 16)]
            for jj in range(16):
                e = 16 * g + jj
                g0 = g0vec[jj]
                g1 = g1vec[jj]
                for j in range(_CH // 16):
                    v0 = rows2[par, e, pl.ds(16 * j, 16)]
                    v1 = rows2[par, e, pl.ds(_CH + 16 * j, 16)]
                    msg2[par, e, pl.ds(16 * j, 16)] = v0 * g0 + v1 * g1

    for ci in range(4):
        pltpu.sync_copy(zeros_hbm.at[:, pl.ds(0, _CH)],
                        acc.at[pl.ds(sid * _RPT, _RPT)])
        plsc.subcore_barrier()
        _edge_pipeline(nb, hts[ci], srcv, dstv, rows2, msg2, acc,
                       gs0, gs1, ss0, ss1, compute_block)
        plsc.subcore_barrier()
        pltpu.sync_copy(acc.at[pl.ds(sid * _RPT, _RPT)],
                        out_hbm.at[cid, ci, pl.ds(sid * _RPT, _RPT)])


_B0_4, _B1_4 = 54, 26
_sc4_call = functools.partial(
    pl.kernel,
    out_type=jax.ShapeDtypeStruct((_NC, 4, _NPAD, _CH), F32),
    mesh=_mesh,
    compiler_params=_sc_params,
    scratch_types=[
        pltpu.VMEM((_B0_4, _EB4), I32),
        pltpu.VMEM((_B0_4, _EB4), I32),
        pltpu.VMEM((_B0_4, _EB4), F32),
        pltpu.VMEM((_B0_4, _EB4), F32),
        pltpu.VMEM((2, _EB4, 2 * _CH), F32),
        pltpu.VMEM((2, _EB4, _CH), F32),
        pltpu.VMEM_SHARED((_NPAD, _CH), F32),
        pltpu.SemaphoreType.DMA,
        pltpu.SemaphoreType.DMA,
        pltpu.SemaphoreType.DMA,
        pltpu.SemaphoreType.DMA,
    ],
)(functools.partial(_sc4_body, _B0_4, _B1_4, _B0_4))


_BN = 1024


def _tc1_body(a_ref, w_ref, b_ref, x_ref, invd_ref):
    a = a_ref[0] + a_ref[1]
    a6 = jnp.concatenate([a[:, 0:3], a[:, 16:19]], axis=1)
    s = jnp.dot(a6, w_ref[...], preferred_element_type=F32)
    deg = a[:, 19:20]
    inv = 1.0 / jnp.maximum(deg, 1.0)
    x_ref[...] = s * inv + b_ref[...]
    invd_ref[...] = jnp.broadcast_to(inv, (_BN, 8))


def _tc1_call(a1, wstk, bias):
    return pl.pallas_call(
        _tc1_body,
        grid=(_NPAD // _BN,),
        in_specs=[
            pl.BlockSpec((_NC, _BN, 32), lambda i: (0, i, 0)),
            pl.BlockSpec((6, 64), lambda i: (0, 0)),
            pl.BlockSpec((1, 64), lambda i: (0, 0)),
        ],
        out_specs=[
            pl.BlockSpec((_BN, 64), lambda i: (i, 0)),
            pl.BlockSpec((_BN, 8), lambda i: (i, 0)),
        ],
        out_shape=[
            jax.ShapeDtypeStruct((_N, 64), F32),
            jax.ShapeDtypeStruct((_N, 8), F32),
        ],
        compiler_params=pltpu.CompilerParams(
            dimension_semantics=("parallel",)),
    )(a1, wstk, bias)


def _tc2_body(a_ref, w_ref, invd_ref, b_ref, o0_ref, o1_ref):
    a = a_ref[0, 0] + a_ref[1, 0]
    s = jnp.dot(a, w_ref[0], preferred_element_type=F32)
    res = s * invd_ref[:, 0:1] + b_ref[...]
    o0_ref[...] = res[:, 0:64]
    o1_ref[...] = res[:, 64:128]


def _tc2_call(a, wstk, invd, bias):
    return pl.pallas_call(
        _tc2_body,
        grid=(_NPAD // _BN,),
        in_specs=[
            pl.BlockSpec((_NC, 1, _BN, 128), lambda i: (0, 0, i, 0)),
            pl.BlockSpec((1, 128, 128), lambda i: (0, 0, 0)),
            pl.BlockSpec((_BN, 8), lambda i: (i, 0)),
            pl.BlockSpec((1, 128), lambda i: (0, 0)),
        ],
        out_specs=[pl.BlockSpec((_BN, 64), lambda i: (i, 0))
                   for _ in range(2)],
        out_shape=[jax.ShapeDtypeStruct((_N, 64), F32) for _ in range(2)],
        compiler_params=pltpu.CompilerParams(
            dimension_semantics=("parallel",)),
    )(a, wstk, invd, bias)


def _tc3_body(a_ref, w_ref, invd_ref, b_ref, *rest):
    ci = pl.program_id(1)
    out_refs = rest[:4]
    accr = rest[4]
    a = a_ref[0, 0] + a_ref[1, 0]
    p = jnp.dot(a, w_ref[0], preferred_element_type=F32)

    @pl.when(ci == 0)
    def _():
        accr[...] = p

    @pl.when(ci > 0)
    def _():
        accr[...] += p

    @pl.when(ci == 1)
    def _():
        x4 = accr[...] * invd_ref[:, 0:1] + b_ref[...]
        for co in range(4):
            out_refs[co][...] = x4[:, 64 * co:64 * (co + 1)]


def _tc3_call(a, wstk, invd, bias):
    return pl.pallas_call(
        _tc3_body,
        grid=(_NPAD // _BN, 2),
        in_specs=[
            pl.BlockSpec((_NC, 1, _BN, 128), lambda i, ci: (0, ci, i, 0)),
            pl.BlockSpec((1, 128, 256), lambda i, ci: (ci, 0, 0)),
            pl.BlockSpec((_BN, 8), lambda i, ci: (i, 0)),
            pl.BlockSpec((1, 256), lambda i, ci: (0, 0)),
        ],
        out_specs=[pl.BlockSpec((_BN, 64), lambda i, ci: (i, 0))
                   for _ in range(4)],
        out_shape=[jax.ShapeDtypeStruct((_N, 64), F32) for _ in range(4)],
        scratch_shapes=[pltpu.VMEM((_BN, 256), F32)],
        compiler_params=pltpu.CompilerParams(
            dimension_semantics=("parallel", "arbitrary")),
    )(a, wstk, invd, bias)


def _tc4_body(a_ref, w_ref, invd_ref, b_ref, out_ref, accr):
    ci = pl.program_id(1)
    a = a_ref[0, 0] + a_ref[1, 0]
    p = jnp.dot(a, w_ref[0], preferred_element_type=F32)

    @pl.when(ci == 0)
    def _():
        accr[...] = p

    @pl.when(ci > 0)
    def _():
        accr[...] += p

    @pl.when(ci == 3)
    def _():
        out_ref[...] = accr[...] * invd_ref[:, 0:1] + b_ref[...]


def _tc4_call(a, wstk, invd, bias):
    return pl.pallas_call(
        _tc4_body,
        grid=(_NPAD // _BN, 4),
        in_specs=[
            pl.BlockSpec((_NC, 1, _BN, 128), lambda i, ci: (0, ci, i, 0)),
            pl.BlockSpec((1, 128, 256), lambda i, ci: (ci, 0, 0)),
            pl.BlockSpec((_BN, 8), lambda i, ci: (i, 0)),
            pl.BlockSpec((1, 256), lambda i, ci: (0, 0)),
        ],
        out_specs=pl.BlockSpec((_BN, 256), lambda i, ci: (i, 0)),
        out_shape=jax.ShapeDtypeStruct((_N, 256), F32),
        scratch_shapes=[pltpu.VMEM((_BN, 256), F32)],
        compiler_params=pltpu.CompilerParams(
            dimension_semantics=("parallel", "arbitrary")),
    )(a, wstk, invd, bias)


def _stack_w(W, out_d):
    in_d = W.shape[0]
    nc = in_d // _CH
    w0 = W[:, :out_d].reshape(nc, _CH, out_d)
    w1 = W[:, out_d:].reshape(nc, _CH, out_d)
    return jnp.concatenate([w0, w1], axis=1)


def kernel(features, edge_index, edge_attr, W1, mu1, is1, b1, W2, mu2, is2,
           b2, W3, mu3, is3, b3, W4, mu4, is4, b4):
    src2d = jnp.pad(edge_index[0], (0, _EPAD - _E)).reshape(_NBLK_P, _EB)
    dst2d = jnp.pad(edge_index[1], (0, _EPAD - _E)).reshape(_NBLK_P, _EB)
    attr_t = jnp.pad(edge_attr, ((0, _EPAD - _E), (0, 0))).T
    feat16 = jnp.pad(features, ((0, 0), (0, 13)))
    mus = jnp.stack([mu1, mu2, mu3, mu4])
    iss = jnp.stack([is1, is2, is3, is4])
    zeros = jnp.zeros((_RPT, 2 * _CH), F32)

    gw = _gw_call(attr_t, mus, iss).reshape(16, _NBLK_P, _EB)

    a1 = _sc1_call(feat16, src2d, dst2d, gw, zeros)
    wstk1 = jnp.concatenate([W1[:, :64], W1[:, 64:]], axis=0)
    x2, invd = _tc1_call(a1, wstk1, b1.reshape(1, 64))

    a2 = _scl_call(1, 1)(x2, src2d, dst2d, gw, zeros)
    x3 = _tc2_call(a2, _stack_w(W2, 128), invd, b2.reshape(1, 128))

    a3 = _scl_call(2, 2)(*x3, src2d, dst2d, gw, zeros)
    x4 = _tc3_call(a3, _stack_w(W3, 256), invd, b3.reshape(1, 256))

    a4 = _scl_call(3, 4)(*x4, src2d, dst2d, gw, zeros)
    return _tc4_call(a4, _stack_w(W4, 256), invd, b4.reshape(1, 256))


# speedup vs baseline: 1.4741x; 1.0168x over previous
import functools

import jax
import jax.numpy as jnp
from jax import lax
from jax.experimental import pallas as pl
from jax.experimental.pallas import tpu as pltpu
from jax.experimental.pallas import tpu_sc as plsc

F32 = jnp.float32
I32 = jnp.int32

_N = 10000
_E = 160000
_K = 2
_NC = 2
_NS = 16
_NW = _NC * _NS
_EB = 64
_BPW = 80
_NBLK = _NW * _BPW
_NBLK_P = 2624
_EPAD = _NBLK_P * _EB
_CH = 64
_NPAD = 10240
_RPT = _NPAD // _NS

_mesh = plsc.VectorSubcoreMesh(
    core_axis_name="c", subcore_axis_name="s", num_cores=_NC, num_subcores=_NS)
_sc_params = pltpu.CompilerParams(use_tc_tiling_on_sc=False)


_GWB = 2048


def _gw_body(attr_ref, mus_ref, iss_ref, out_ref):
    i = pl.program_id(0)
    eidx = i * _GWB + lax.broadcasted_iota(I32, (1, _GWB), 1)
    valid = (eidx < _E).astype(F32)
    rows = []
    for l in range(4):
        for k in range(_K):
            q = jnp.zeros((1, _GWB), F32)
            for p in range(3):
                d = (attr_ref[p:p + 1, :] - mus_ref[l, k, p]) * iss_ref[l, k, p]
                q = q + d * d
            rows.append(jnp.exp(-0.5 * q) * valid)
    rows.append(valid)
    for _ in range(7):
        rows.append(jnp.zeros((1, _GWB), F32))
    out_ref[...] = jnp.concatenate(rows, axis=0)


def _gw_call(attr_t, mus, iss):
    return pl.pallas_call(
        _gw_body,
        grid=(_EPAD // _GWB,),
        in_specs=[
            pl.BlockSpec((3, _GWB), lambda i: (0, i)),
            pl.BlockSpec(memory_space=pltpu.SMEM),
            pl.BlockSpec(memory_space=pltpu.SMEM),
        ],
        out_specs=pl.BlockSpec((16, _GWB), lambda i: (0, i)),
        out_shape=jax.ShapeDtypeStruct((16, _EPAD), F32),
        compiler_params=pltpu.CompilerParams(
            dimension_semantics=("parallel",)),
    )(attr_t, mus, iss)


def _edge_pipeline(nb, table, srcv, dstv, rows2, msg2, acc,
                   gs0, gs1, ss0, ss1, compute_block):
    gsems = (gs0, gs1)
    ssems = (ss0, ss1)

    def g_start(b, par):
        pltpu.make_async_copy(
            table.at[srcv.at[b]], rows2.at[par], gsems[par]).start()

    def g_wait(b, par):
        pltpu.make_async_copy(
            table.at[srcv.at[b]], rows2.at[par], gsems[par]).wait()

    def s_start(b, par):
        pltpu.async_copy(
            msg2.at[par], acc.at[dstv.at[b]], ssems[par], add=True)

    def s_wait(b, par):
        pltpu.make_async_copy(
            msg2.at[par], acc.at[dstv.at[b]], ssems[par]).wait()

    g_start(0, 0)
    g_start(1, 1)

    @pl.loop(0, nb // 2)
    def _pair(bb):
        for par in (0, 1):
            b = 2 * bb + par
            g_wait(b, par)

            @pl.when(b >= 2)
            def _():
                s_wait(b - 2, par)

            compute_block(b, par)

            @pl.when(b + 2 < nb)
            def _():
                g_start(b + 2, par)

            s_start(b, par)

    s_wait(nb - 2, 0)
    s_wait(nb - 1, 1)


def _sc1_body(B0, B1, BMAX, feat_hbm, src_hbm, dst_hbm, gw_hbm, zeros_hbm,
              out_hbm, srcv, dstv, g0v, g1v, wv, rows2, msg2, acc,
              gs0, gs1, ss0, ss1):
    cid = lax.axis_index("c")
    sid = lax.axis_index("s")
    nb = jnp.where(cid == 0, B0, B1)
    base = jnp.where(cid == 0, sid * B0, _NS * B0 + sid * B1)
    lane = lax.broadcasted_iota(I32, (16,), 0)

    pltpu.sync_copy(zeros_hbm.at[:, pl.ds(0, 32)],
                    acc.at[pl.ds(sid * _RPT, _RPT)])
    pltpu.sync_copy(src_hbm.at[pl.ds(base, BMAX)], srcv)
    pltpu.sync_copy(dst_hbm.at[pl.ds(base, BMAX)], dstv)
    pltpu.sync_copy(gw_hbm.at[0, pl.ds(base, BMAX)], g0v)
    pltpu.sync_copy(gw_hbm.at[1, pl.ds(base, BMAX)], g1v)
    pltpu.sync_copy(gw_hbm.at[8, pl.ds(base, BMAX)], wv)
    plsc.subcore_barrier()

    def compute_block(b, par):
        @pl.loop(0, _EB // 16)
        def _grp(g):
            g0vec = g0v[b, pl.ds(16 * g, 16)]
            g1vec = g1v[b, pl.ds(16 * g, 16)]
            wvec = wv[b, pl.ds(16 * g, 16)]
            for j in range(16):
                e = 16 * g + j
                v = rows2[par, e, :]
                m0 = jnp.where(lane < 3, v * g0vec[j], 0.0)
                m1 = jnp.where(lane < 3, v * g1vec[j],
                               jnp.where(lane == 3, wvec[j], 0.0))
                msg2[par, e, pl.ds(0, 16)] = m0
                msg2[par, e, pl.ds(16, 16)] = m1

    _edge_pipeline(nb, feat_hbm, srcv, dstv, rows2, msg2, acc,
                   gs0, gs1, ss0, ss1, compute_block)

    plsc.subcore_barrier()
    pltpu.sync_copy(acc.at[pl.ds(sid * _RPT, _RPT)],
                    out_hbm.at[cid, pl.ds(sid * _RPT, _RPT)])


_B0_1, _B1_1 = 84, 76
_sc1_call = functools.partial(
    pl.kernel,
    out_type=jax.ShapeDtypeStruct((_NC, _NPAD, 32), F32),
    mesh=_mesh,
    compiler_params=_sc_params,
    scratch_types=[
        pltpu.VMEM((_B0_1, _EB), I32),
        pltpu.VMEM((_B0_1, _EB), I32),
        pltpu.VMEM((_B0_1, _EB), F32),
        pltpu.VMEM((_B0_1, _EB), F32),
        pltpu.VMEM((_B0_1, _EB), F32),
        pltpu.VMEM((2, _EB, 16), F32),
        pltpu.VMEM((2, _EB, 32), F32),
        pltpu.VMEM_SHARED((_NPAD, 32), F32),
        pltpu.SemaphoreType.DMA,
        pltpu.SemaphoreType.DMA,
        pltpu.SemaphoreType.DMA,
        pltpu.SemaphoreType.DMA,
    ],
)(functools.partial(_sc1_body, _B0_1, _B1_1, _B0_1))


def _scl_body(layer, nc, B0, B1, BMAX, *refs):
    xcs = refs[:nc]
    (src_hbm, dst_hbm, gw_hbm, zeros_hbm, out_hbm,
     srcv, dstv, g0v, g1v, rows2, msg2, acc, gs0, gs1, ss0, ss1) = refs[nc:]
    cid = lax.axis_index("c")
    sid = lax.axis_index("s")
    nb = jnp.where(cid == 0, B0, B1)
    base = jnp.where(cid == 0, sid * B0, _NS * B0 + sid * B1)

    pltpu.sync_copy(src_hbm.at[pl.ds(base, BMAX)], srcv)
    pltpu.sync_copy(dst_hbm.at[pl.ds(base, BMAX)], dstv)
    pltpu.sync_copy(gw_hbm.at[2 * layer, pl.ds(base, BMAX)], g0v)
    pltpu.sync_copy(gw_hbm.at[2 * layer + 1, pl.ds(base, BMAX)], g1v)

    def compute_block(b, par):
        @pl.loop(0, _EB // 16)
        def _grp(g):
            g0vec = g0v[b, pl.ds(16 * g, 16)]
            g1vec = g1v[b, pl.ds(16 * g, 16)]
            for jj in range(16):
                e = 16 * g + jj
                g0 = g0vec[jj]
                g1 = g1vec[jj]
                for j in range(_CH // 16):
                    v = rows2[par, e, pl.ds(16 * j, 16)]
                    msg2[par, e, pl.ds(16 * j, 16)] = v * g0
                    msg2[par, e, pl.ds(_CH + 16 * j, 16)] = v * g1

    for ci in range(nc):
        pltpu.sync_copy(zeros_hbm, acc.at[pl.ds(sid * _RPT, _RPT)])
        plsc.subcore_barrier()
        _edge_pipeline(nb, xcs[ci], srcv, dstv, rows2, msg2, acc,
                       gs0, gs1, ss0, ss1, compute_block)
        plsc.subcore_barrier()
        pltpu.sync_copy(acc.at[pl.ds(sid * _RPT, _RPT)],
                        out_hbm.at[cid, ci, pl.ds(sid * _RPT, _RPT)])


_B0_23, _B1_23 = 94, 66


def _scl_call(layer, nc, B0=_B0_23, B1=_B1_23):
    return functools.partial(
        pl.kernel,
        out_type=jax.ShapeDtypeStruct((_NC, nc, _NPAD, 2 * _CH), F32),
        mesh=_mesh,
        compiler_params=_sc_params,
        scratch_types=[
            pltpu.VMEM((_B0_23, _EB), I32),
            pltpu.VMEM((_B0_23, _EB), I32),
            pltpu.VMEM((_B0_23, _EB), F32),
            pltpu.VMEM((_B0_23, _EB), F32),
            pltpu.VMEM((2, _EB, _CH), F32),
            pltpu.VMEM((2, _EB, 2 * _CH), F32),
            pltpu.VMEM_SHARED((_NPAD, 2 * _CH), F32),
            pltpu.SemaphoreType.DMA,
            pltpu.SemaphoreType.DMA,
            pltpu.SemaphoreType.DMA,
            pltpu.SemaphoreType.DMA,
        ],
    )(functools.partial(_scl_body, layer, nc, B0, B1, _B0_23))


_EB4 = 128


def _sc4_body(B0, B1, BMAX, h0, h1, h2, h3, src_hbm, dst_hbm, gw_hbm,
              zeros_hbm, out_hbm, srcv, dstv, g0v, g1v, rows2, msg2, acc,
              gs0, gs1, ss0, ss1):
    hts = (h0, h1, h2, h3)
    cid = lax.axis_index("c")
    sid = lax.axis_index("s")
    nb = jnp.where(cid == 0, B0, B1)
    base = jnp.where(cid == 0, sid * B0, _NS * B0 + sid * B1)

    pltpu.sync_copy(src_hbm.at[pl.ds(base, BMAX)], srcv)
    pltpu.sync_copy(dst_hbm.at[pl.ds(base, BMAX)], dstv)
    pltpu.sync_copy(gw_hbm.at[6, pl.ds(base, BMAX)], g0v)
    pltpu.sync_copy(gw_hbm.at[7, pl.ds(base, BMAX)], g1v)

    def compute_block(b, par):
        @pl.loop(0, _EB4 // 16)
        def _grp(g):
            g0vec = g0v[b, pl.ds(16 * g, 16)]
            g1vec = g1v[b, pl.ds(16 * g, 16)]
            for jj in range(16):
                e = 16 * g + jj
                g0 = g0vec[jj]
                g1 = g1vec[jj]
                for j in range(_CH // 16):
                    v0 = rows2[par, e, pl.ds(16 * j, 16)]
                    v1 = rows2[par, e, pl.ds(_CH + 16 * j, 16)]
                    msg2[par, e, pl.ds(16 * j, 16)] = v0 * g0 + v1 * g1

    for ci in range(4):
        pltpu.sync_copy(zeros_hbm.at[:, pl.ds(0, _CH)],
                        acc.at[pl.ds(sid * _RPT, _RPT)])
        plsc.subcore_barrier()
        _edge_pipeline(nb, hts[ci], srcv, dstv, rows2, msg2, acc,
                       gs0, gs1, ss0, ss1, compute_block)
        plsc.subcore_barrier()
        pltpu.sync_copy(acc.at[pl.ds(sid * _RPT, _RPT)],
                        out_hbm.at[cid, ci, pl.ds(sid * _RPT, _RPT)])


_B0_4, _B1_4 = 54, 26
_sc4_call = functools.partial(
    pl.kernel,
    out_type=jax.ShapeDtypeStruct((_NC, 4, _NPAD, _CH), F32),
    mesh=_mesh,
    compiler_params=_sc_params,
    scratch_types=[
        pltpu.VMEM((_B0_4, _EB4), I32),
        pltpu.VMEM((_B0_4, _EB4), I32),
        pltpu.VMEM((_B0_4, _EB4), F32),
        pltpu.VMEM((_B0_4, _EB4), F32),
        pltpu.VMEM((2, _EB4, 2 * _CH), F32),
        pltpu.VMEM((2, _EB4, _CH), F32),
        pltpu.VMEM_SHARED((_NPAD, _CH), F32),
        pltpu.SemaphoreType.DMA,
        pltpu.SemaphoreType.DMA,
        pltpu.SemaphoreType.DMA,
        pltpu.SemaphoreType.DMA,
    ],
)(functools.partial(_sc4_body, _B0_4, _B1_4, _B0_4))


_BN = 1024


def _tc1_body(a_ref, w_ref, b_ref, x_ref, invd_ref):
    a = a_ref[0] + a_ref[1]
    a6 = jnp.concatenate([a[:, 0:3], a[:, 16:19]], axis=1)
    s = jnp.dot(a6, w_ref[...], preferred_element_type=F32)
    deg = a[:, 19:20]
    inv = 1.0 / jnp.maximum(deg, 1.0)
    x_ref[...] = s * inv + b_ref[...]
    invd_ref[...] = jnp.broadcast_to(inv, (_BN, 8))


def _tc1_call(a1, wstk, bias):
    return pl.pallas_call(
        _tc1_body,
        grid=(_NPAD // _BN,),
        in_specs=[
            pl.BlockSpec((_NC, _BN, 32), lambda i: (0, i, 0)),
            pl.BlockSpec((6, 64), lambda i: (0, 0)),
            pl.BlockSpec((1, 64), lambda i: (0, 0)),
        ],
        out_specs=[
            pl.BlockSpec((_BN, 64), lambda i: (i, 0)),
            pl.BlockSpec((_BN, 8), lambda i: (i, 0)),
        ],
        out_shape=[
            jax.ShapeDtypeStruct((_N, 64), F32),
            jax.ShapeDtypeStruct((_N, 8), F32),
        ],
        compiler_params=pltpu.CompilerParams(
            dimension_semantics=("parallel",)),
    )(a1, wstk, bias)


def _tc2_body(a_ref, w_ref, invd_ref, b_ref, o0_ref, o1_ref):
    a = a_ref[0, 0] + a_ref[1, 0]
    s = jnp.dot(a, w_ref[0], preferred_element_type=F32)
    res = s * invd_ref[:, 0:1] + b_ref[...]
    o0_ref[...] = res[:, 0:64]
    o1_ref[...] = res[:, 64:128]


def _tc2_call(a, wstk, invd, bias):
    return pl.pallas_call(
        _tc2_body,
        grid=(_NPAD // _BN,),
        in_specs=[
            pl.BlockSpec((_NC, 1, _BN, 128), lambda i: (0, 0, i, 0)),
            pl.BlockSpec((1, 128, 128), lambda i: (0, 0, 0)),
            pl.BlockSpec((_BN, 8), lambda i: (i, 0)),
            pl.BlockSpec((1, 128), lambda i: (0, 0)),
        ],
        out_specs=[pl.BlockSpec((_BN, 64), lambda i: (i, 0))
                   for _ in range(2)],
        out_shape=[jax.ShapeDtypeStruct((_N, 64), F32) for _ in range(2)],
        compiler_params=pltpu.CompilerParams(
            dimension_semantics=("parallel",)),
    )(a, wstk, invd, bias)


def _tc3_body(a_ref, w_ref, invd_ref, b_ref, *rest):
    ci = pl.program_id(1)
    out_refs = rest[:4]
    accr = rest[4]
    a = a_ref[0, 0] + a_ref[1, 0]
    p = jnp.dot(a, w_ref[0], preferred_element_type=F32)

    @pl.when(ci == 0)
    def _():
        accr[...] = p

    @pl.when(ci > 0)
    def _():
        accr[...] += p

    @pl.when(ci == 1)
    def _():
        x4 = accr[...] * invd_ref[:, 0:1] + b_ref[...]
        for co in range(4):
            out_refs[co][...] = x4[:, 64 * co:64 * (co + 1)]


def _tc3_call(a, wstk, invd, bias):
    return pl.pallas_call(
        _tc3_body,
        grid=(_NPAD // _BN, 2),
        in_specs=[
            pl.BlockSpec((_NC, 1, _BN, 128), lambda i, ci: (0, ci, i, 0)),
            pl.BlockSpec((1, 128, 256), lambda i, ci: (ci, 0, 0)),
            pl.BlockSpec((_BN, 8), lambda i, ci: (i, 0)),
            pl.BlockSpec((1, 256), lambda i, ci: (0, 0)),
        ],
        out_specs=[pl.BlockSpec((_BN, 64), lambda i, ci: (i, 0))
                   for _ in range(4)],
        out_shape=[jax.ShapeDtypeStruct((_N, 64), F32) for _ in range(4)],
        scratch_shapes=[pltpu.VMEM((_BN, 256), F32)],
        compiler_params=pltpu.CompilerParams(
            dimension_semantics=("parallel", "arbitrary")),
    )(a, wstk, invd, bias)


def _tc4_body(a_ref, w_ref, invd_ref, b_ref, out_ref, accr):
    ci = pl.program_id(1)
    a = a_ref[0, 0] + a_ref[1, 0]
    p = jnp.dot(a, w_ref[0], preferred_element_type=F32)

    @pl.when(ci == 0)
    def _():
        accr[...] = p

    @pl.when(ci > 0)
    def _():
        accr[...] += p

    @pl.when(ci == 3)
    def _():
        out_ref[...] = accr[...] * invd_ref[:, 0:1] + b_ref[...]


def _tc4_call(a, wstk, invd, bias):
    return pl.pallas_call(
        _tc4_body,
        grid=(_NPAD // _BN, 4),
        in_specs=[
            pl.BlockSpec((_NC, 1, _BN, 128), lambda i, ci: (0, ci, i, 0)),
            pl.BlockSpec((1, 128, 256), lambda i, ci: (ci, 0, 0)),
            pl.BlockSpec((_BN, 8), lambda i, ci: (i, 0)),
            pl.BlockSpec((1, 256), lambda i, ci: (0, 0)),
        ],
        out_specs=pl.BlockSpec((_BN, 256), lambda i, ci: (i, 0)),
        out_shape=jax.ShapeDtypeStruct((_N, 256), F32),
        scratch_shapes=[pltpu.VMEM((_BN, 256), F32)],
        compiler_params=pltpu.CompilerParams(
            dimension_semantics=("parallel", "arbitrary")),
    )(a, wstk, invd, bias)


def _stack_w(W, out_d):
    in_d = W.shape[0]
    nc = in_d // _CH
    w0 = W[:, :out_d].reshape(nc, _CH, out_d)
    w1 = W[:, out_d:].reshape(nc, _CH, out_d)
    return jnp.concatenate([w0, w1], axis=1)


def kernel(features, edge_index, edge_attr, W1, mu1, is1, b1, W2, mu2, is2,
           b2, W3, mu3, is3, b3, W4, mu4, is4, b4):
    src2d = jnp.pad(edge_index[0], (0, _EPAD - _E)).reshape(_NBLK_P, _EB)
    dst2d = jnp.pad(edge_index[1], (0, _EPAD - _E)).reshape(_NBLK_P, _EB)
    attr_t = jnp.pad(edge_attr, ((0, _EPAD - _E), (0, 0))).T
    feat16 = jnp.pad(features, ((0, 0), (0, 13)))
    mus = jnp.stack([mu1, mu2, mu3, mu4])
    iss = jnp.stack([is1, is2, is3, is4])
    zeros = jnp.zeros((_RPT, 2 * _CH), F32)

    gw = _gw_call(attr_t, mus, iss).reshape(16, _NBLK_P, _EB)

    a1 = _sc1_call(feat16, src2d, dst2d, gw, zeros)
    wstk1 = jnp.concatenate([W1[:, :64], W1[:, 64:]], axis=0)
    x2, invd = _tc1_call(a1, wstk1, b1.reshape(1, 64))

    a2 = _scl_call(1, 1)(x2, src2d, dst2d, gw, zeros)
    x3 = _tc2_call(a2, _stack_w(W2, 128), invd, b2.reshape(1, 128))

    a3 = _scl_call(2, 2, 92, 68)(*x3, src2d, dst2d, gw, zeros)
    x4 = _tc3_call(a3, _stack_w(W3, 256), invd, b3.reshape(1, 256))

    a4 = _scl_call(3, 4, 90, 70)(*x4, src2d, dst2d, gw, zeros)
    return _tc4_call(a4, _stack_w(W4, 256), invd, b4.reshape(1, 256))


# speedup vs baseline: 1.4752x; 1.0008x over previous
import functools

import jax
import jax.numpy as jnp
from jax import lax
from jax.experimental import pallas as pl
from jax.experimental.pallas import tpu as pltpu
from jax.experimental.pallas import tpu_sc as plsc

F32 = jnp.float32
I32 = jnp.int32

_N = 10000
_E = 160000
_K = 2
_NC = 2
_NS = 16
_NW = _NC * _NS
_EB = 64
_BPW = 80
_NBLK = _NW * _BPW
_NBLK_P = 2624
_EPAD = _NBLK_P * _EB
_CH = 64
_NPAD = 10240
_RPT = _NPAD // _NS

_mesh = plsc.VectorSubcoreMesh(
    core_axis_name="c", subcore_axis_name="s", num_cores=_NC, num_subcores=_NS)
_sc_params = pltpu.CompilerParams(use_tc_tiling_on_sc=False)


_GWB = 2048


def _gw_body(attr_ref, mus_ref, iss_ref, out_ref):
    i = pl.program_id(0)
    eidx = i * _GWB + lax.broadcasted_iota(I32, (1, _GWB), 1)
    valid = (eidx < _E).astype(F32)
    rows = []
    for l in range(4):
        for k in range(_K):
            q = jnp.zeros((1, _GWB), F32)
            for p in range(3):
                d = (attr_ref[p:p + 1, :] - mus_ref[l, k, p]) * iss_ref[l, k, p]
                q = q + d * d
            rows.append(jnp.exp(-0.5 * q) * valid)
    rows.append(valid)
    for _ in range(7):
        rows.append(jnp.zeros((1, _GWB), F32))
    out_ref[...] = jnp.concatenate(rows, axis=0)


def _gw_call(attr_t, mus, iss):
    return pl.pallas_call(
        _gw_body,
        grid=(_EPAD // _GWB,),
        in_specs=[
            pl.BlockSpec((3, _GWB), lambda i: (0, i)),
            pl.BlockSpec(memory_space=pltpu.SMEM),
            pl.BlockSpec(memory_space=pltpu.SMEM),
        ],
        out_specs=pl.BlockSpec((16, _GWB), lambda i: (0, i)),
        out_shape=jax.ShapeDtypeStruct((16, _EPAD), F32),
        compiler_params=pltpu.CompilerParams(
            dimension_semantics=("parallel",)),
    )(attr_t, mus, iss)


def _edge_pipeline(nb, table, srcv, dstv, rows2, msg2, acc,
                   gs0, gs1, ss0, ss1, compute_block):
    gsems = (gs0, gs1)
    ssems = (ss0, ss1)

    def g_start(b, par):
        pltpu.make_async_copy(
            table.at[srcv.at[b]], rows2.at[par], gsems[par]).start()

    def g_wait(b, par):
        pltpu.make_async_copy(
            table.at[srcv.at[b]], rows2.at[par], gsems[par]).wait()

    def s_start(b, par):
        pltpu.async_copy(
            msg2.at[par], acc.at[dstv.at[b]], ssems[par], add=True)

    def s_wait(b, par):
        pltpu.make_async_copy(
            msg2.at[par], acc.at[dstv.at[b]], ssems[par]).wait()

    g_start(0, 0)
    g_start(1, 1)

    @pl.loop(0, nb // 2)
    def _pair(bb):
        for par in (0, 1):
            b = 2 * bb + par
            g_wait(b, par)

            @pl.when(b >= 2)
            def _():
                s_wait(b - 2, par)

            compute_block(b, par)

            @pl.when(b + 2 < nb)
            def _():
                g_start(b + 2, par)

            s_start(b, par)

    s_wait(nb - 2, 0)
    s_wait(nb - 1, 1)


def _sc1_body(B0, B1, BMAX, feat_hbm, src_hbm, dst_hbm, gw_hbm, zeros_hbm,
              out_hbm, srcv, dstv, g0v, g1v, wv, rows2, msg2, acc,
              gs0, gs1, ss0, ss1):
    cid = lax.axis_index("c")
    sid = lax.axis_index("s")
    nb = jnp.where(cid == 0, B0, B1)
    base = jnp.where(cid == 0, sid * B0, _NS * B0 + sid * B1)
    lane = lax.broadcasted_iota(I32, (16,), 0)

    pltpu.sync_copy(zeros_hbm.at[:, pl.ds(0, 32)],
                    acc.at[pl.ds(sid * _RPT, _RPT)])
    pltpu.sync_copy(src_hbm.at[pl.ds(base, BMAX)], srcv)
    pltpu.sync_copy(dst_hbm.at[pl.ds(base, BMAX)], dstv)
    pltpu.sync_copy(gw_hbm.at[0, pl.ds(base, BMAX)], g0v)
    pltpu.sync_copy(gw_hbm.at[1, pl.ds(base, BMAX)], g1v)
    pltpu.sync_copy(gw_hbm.at[8, pl.ds(base, BMAX)], wv)
    plsc.subcore_barrier()

    def compute_block(b, par):
        @pl.loop(0, _EB // 16)
        def _grp(g):
            g0vec = g0v[b, pl.ds(16 * g, 16)]
            g1vec = g1v[b, pl.ds(16 * g, 16)]
            wvec = wv[b, pl.ds(16 * g, 16)]
            for j in range(16):
                e = 16 * g + j
                v = rows2[par, e, :]
                m0 = jnp.where(lane < 3, v * g0vec[j], 0.0)
                m1 = jnp.where(lane < 3, v * g1vec[j],
                               jnp.where(lane == 3, wvec[j], 0.0))
                msg2[par, e, pl.ds(0, 16)] = m0
                msg2[par, e, pl.ds(16, 16)] = m1

    _edge_pipeline(nb, feat_hbm, srcv, dstv, rows2, msg2, acc,
                   gs0, gs1, ss0, ss1, compute_block)

    plsc.subcore_barrier()
    pltpu.sync_copy(acc.at[pl.ds(sid * _RPT, _RPT)],
                    out_hbm.at[cid, pl.ds(sid * _RPT, _RPT)])


_B0_1, _B1_1 = 84, 76
_sc1_call = functools.partial(
    pl.kernel,
    out_type=jax.ShapeDtypeStruct((_NC, _NPAD, 32), F32),
    mesh=_mesh,
    compiler_params=_sc_params,
    scratch_types=[
        pltpu.VMEM((_B0_1, _EB), I32),
        pltpu.VMEM((_B0_1, _EB), I32),
        pltpu.VMEM((_B0_1, _EB), F32),
        pltpu.VMEM((_B0_1, _EB), F32),
        pltpu.VMEM((_B0_1, _EB), F32),
        pltpu.VMEM((2, _EB, 16), F32),
        pltpu.VMEM((2, _EB, 32), F32),
        pltpu.VMEM_SHARED((_NPAD, 32), F32),
        pltpu.SemaphoreType.DMA,
        pltpu.SemaphoreType.DMA,
        pltpu.SemaphoreType.DMA,
        pltpu.SemaphoreType.DMA,
    ],
)(functools.partial(_sc1_body, _B0_1, _B1_1, _B0_1))


def _scl_body(layer, nc, B0, B1, BMAX, *refs):
    xcs = refs[:nc]
    (src_hbm, dst_hbm, gw_hbm, zeros_hbm, out_hbm,
     srcv, dstv, g0v, g1v, rows2, msg2, acc, gs0, gs1, ss0, ss1) = refs[nc:]
    cid = lax.axis_index("c")
    sid = lax.axis_index("s")
    nb = jnp.where(cid == 0, B0, B1)
    base = jnp.where(cid == 0, sid * B0, _NS * B0 + sid * B1)

    pltpu.sync_copy(src_hbm.at[pl.ds(base, BMAX)], srcv)
    pltpu.sync_copy(dst_hbm.at[pl.ds(base, BMAX)], dstv)
    pltpu.sync_copy(gw_hbm.at[2 * layer, pl.ds(base, BMAX)], g0v)
    pltpu.sync_copy(gw_hbm.at[2 * layer + 1, pl.ds(base, BMAX)], g1v)

    def compute_block(b, par):
        @pl.loop(0, _EB // 16)
        def _grp(g):
            g0vec = g0v[b, pl.ds(16 * g, 16)]
            g1vec = g1v[b, pl.ds(16 * g, 16)]
            for jj in range(16):
                e = 16 * g + jj
                g0 = g0vec[jj]
                g1 = g1vec[jj]
                for j in range(_CH // 16):
                    v = rows2[par, e, pl.ds(16 * j, 16)]
                    msg2[par, e, pl.ds(16 * j, 16)] = v * g0
                    msg2[par, e, pl.ds(_CH + 16 * j, 16)] = v * g1

    for ci in range(nc):
        pltpu.sync_copy(zeros_hbm, acc.at[pl.ds(sid * _RPT, _RPT)])
        plsc.subcore_barrier()
        _edge_pipeline(nb, xcs[ci], srcv, dstv, rows2, msg2, acc,
                       gs0, gs1, ss0, ss1, compute_block)
        plsc.subcore_barrier()
        pltpu.sync_copy(acc.at[pl.ds(sid * _RPT, _RPT)],
                        out_hbm.at[cid, ci, pl.ds(sid * _RPT, _RPT)])


_B0_23, _B1_23 = 94, 66


def _scl_call(layer, nc, B0=_B0_23, B1=_B1_23):
    return functools.partial(
        pl.kernel,
        out_type=jax.ShapeDtypeStruct((_NC, nc, _NPAD, 2 * _CH), F32),
        mesh=_mesh,
        compiler_params=_sc_params,
        scratch_types=[
            pltpu.VMEM((_B0_23, _EB), I32),
            pltpu.VMEM((_B0_23, _EB), I32),
            pltpu.VMEM((_B0_23, _EB), F32),
            pltpu.VMEM((_B0_23, _EB), F32),
            pltpu.VMEM((2, _EB, _CH), F32),
            pltpu.VMEM((2, _EB, 2 * _CH), F32),
            pltpu.VMEM_SHARED((_NPAD, 2 * _CH), F32),
            pltpu.SemaphoreType.DMA,
            pltpu.SemaphoreType.DMA,
            pltpu.SemaphoreType.DMA,
            pltpu.SemaphoreType.DMA,
        ],
    )(functools.partial(_scl_body, layer, nc, B0, B1, _B0_23))


_BN = 1024


def _tc1_body(a_ref, w_ref, b_ref, x_ref, invd_ref):
    a = a_ref[0] + a_ref[1]
    a6 = jnp.concatenate([a[:, 0:3], a[:, 16:19]], axis=1)
    s = jnp.dot(a6, w_ref[...], preferred_element_type=F32)
    deg = a[:, 19:20]
    inv = 1.0 / jnp.maximum(deg, 1.0)
    x_ref[...] = s * inv + b_ref[...]
    invd_ref[...] = jnp.broadcast_to(inv, (_BN, 8))


def _tc1_call(a1, wstk, bias):
    return pl.pallas_call(
        _tc1_body,
        grid=(_NPAD // _BN,),
        in_specs=[
            pl.BlockSpec((_NC, _BN, 32), lambda i: (0, i, 0)),
            pl.BlockSpec((6, 64), lambda i: (0, 0)),
            pl.BlockSpec((1, 64), lambda i: (0, 0)),
        ],
        out_specs=[
            pl.BlockSpec((_BN, 64), lambda i: (i, 0)),
            pl.BlockSpec((_BN, 8), lambda i: (i, 0)),
        ],
        out_shape=[
            jax.ShapeDtypeStruct((_N, 64), F32),
            jax.ShapeDtypeStruct((_N, 8), F32),
        ],
        compiler_params=pltpu.CompilerParams(
            dimension_semantics=("parallel",)),
    )(a1, wstk, bias)


def _tc2_body(a_ref, w_ref, invd_ref, b_ref, o0_ref, o1_ref):
    a = a_ref[0, 0] + a_ref[1, 0]
    s = jnp.dot(a, w_ref[0], preferred_element_type=F32)
    res = s * invd_ref[:, 0:1] + b_ref[...]
    o0_ref[...] = res[:, 0:64]
    o1_ref[...] = res[:, 64:128]


def _tc2_call(a, wstk, invd, bias):
    return pl.pallas_call(
        _tc2_body,
        grid=(_NPAD // _BN,),
        in_specs=[
            pl.BlockSpec((_NC, 1, _BN, 128), lambda i: (0, 0, i, 0)),
            pl.BlockSpec((1, 128, 128), lambda i: (0, 0, 0)),
            pl.BlockSpec((_BN, 8), lambda i: (i, 0)),
            pl.BlockSpec((1, 128), lambda i: (0, 0)),
        ],
        out_specs=[pl.BlockSpec((_BN, 64), lambda i: (i, 0))
                   for _ in range(2)],
        out_shape=[jax.ShapeDtypeStruct((_N, 64), F32) for _ in range(2)],
        compiler_params=pltpu.CompilerParams(
            dimension_semantics=("parallel",)),
    )(a, wstk, invd, bias)


def _tc3_body(a_ref, w_ref, invd_ref, b_ref, *rest):
    ci = pl.program_id(1)
    out_refs = rest[:4]
    accr = rest[4]
    a = a_ref[0, 0] + a_ref[1, 0]
    p = jnp.dot(a, w_ref[0], preferred_element_type=F32)

    @pl.when(ci == 0)
    def _():
        accr[...] = p

    @pl.when(ci > 0)
    def _():
        accr[...] += p

    @pl.when(ci == 1)
    def _():
        x4 = accr[...] * invd_ref[:, 0:1] + b_ref[...]
        for co in range(4):
            out_refs[co][...] = x4[:, 64 * co:64 * (co + 1)]


def _tc3_call(a, wstk, invd, bias):
    return pl.pallas_call(
        _tc3_body,
        grid=(_NPAD // _BN, 2),
        in_specs=[
            pl.BlockSpec((_NC, 1, _BN, 128), lambda i, ci: (0, ci, i, 0)),
            pl.BlockSpec((1, 128, 256), lambda i, ci: (ci, 0, 0)),
            pl.BlockSpec((_BN, 8), lambda i, ci: (i, 0)),
            pl.BlockSpec((1, 256), lambda i, ci: (0, 0)),
        ],
        out_specs=[pl.BlockSpec((_BN, 64), lambda i, ci: (i, 0))
                   for _ in range(4)],
        out_shape=[jax.ShapeDtypeStruct((_N, 64), F32) for _ in range(4)],
        scratch_shapes=[pltpu.VMEM((_BN, 256), F32)],
        compiler_params=pltpu.CompilerParams(
            dimension_semantics=("parallel", "arbitrary")),
    )(a, wstk, invd, bias)


def _tc4_body(a_ref, w_ref, invd_ref, b_ref, out_ref, accr):
    ci = pl.program_id(1)
    a = a_ref[0, 0] + a_ref[1, 0]
    p = jnp.dot(a, w_ref[0], preferred_element_type=F32)

    @pl.when(ci == 0)
    def _():
        accr[...] = p

    @pl.when(ci > 0)
    def _():
        accr[...] += p

    @pl.when(ci == 3)
    def _():
        out_ref[...] = accr[...] * invd_ref[:, 0:1] + b_ref[...]


def _tc4_call(a, wstk, invd, bias):
    return pl.pallas_call(
        _tc4_body,
        grid=(_NPAD // _BN, 4),
        in_specs=[
            pl.BlockSpec((_NC, 1, _BN, 128), lambda i, ci: (0, ci, i, 0)),
            pl.BlockSpec((1, 128, 256), lambda i, ci: (ci, 0, 0)),
            pl.BlockSpec((_BN, 8), lambda i, ci: (i, 0)),
            pl.BlockSpec((1, 256), lambda i, ci: (0, 0)),
        ],
        out_specs=pl.BlockSpec((_BN, 256), lambda i, ci: (i, 0)),
        out_shape=jax.ShapeDtypeStruct((_N, 256), F32),
        scratch_shapes=[pltpu.VMEM((_BN, 256), F32)],
        compiler_params=pltpu.CompilerParams(
            dimension_semantics=("parallel", "arbitrary")),
    )(a, wstk, invd, bias)


def _stack_w(W, out_d):
    in_d = W.shape[0]
    nc = in_d // _CH
    w0 = W[:, :out_d].reshape(nc, _CH, out_d)
    w1 = W[:, out_d:].reshape(nc, _CH, out_d)
    return jnp.concatenate([w0, w1], axis=1)


def kernel(features, edge_index, edge_attr, W1, mu1, is1, b1, W2, mu2, is2,
           b2, W3, mu3, is3, b3, W4, mu4, is4, b4):
    src2d = jnp.pad(edge_index[0], (0, _EPAD - _E)).reshape(_NBLK_P, _EB)
    dst2d = jnp.pad(edge_index[1], (0, _EPAD - _E)).reshape(_NBLK_P, _EB)
    attr_t = jnp.pad(edge_attr, ((0, _EPAD - _E), (0, 0))).T
    feat16 = jnp.pad(features, ((0, 0), (0, 13)))
    mus = jnp.stack([mu1, mu2, mu3, mu4])
    iss = jnp.stack([is1, is2, is3, is4])
    zeros = jnp.zeros((_RPT, 2 * _CH), F32)

    gw = _gw_call(attr_t, mus, iss).reshape(16, _NBLK_P, _EB)

    a1 = _sc1_call(feat16, src2d, dst2d, gw, zeros)
    wstk1 = jnp.concatenate([W1[:, :64], W1[:, 64:]], axis=0)
    x2, invd = _tc1_call(a1, wstk1, b1.reshape(1, 64))

    a2 = _scl_call(1, 1)(x2, src2d, dst2d, gw, zeros)
    x3 = _tc2_call(a2, _stack_w(W2, 128), invd, b2.reshape(1, 128))

    a3 = _scl_call(2, 2, 92, 68)(*x3, src2d, dst2d, gw, zeros)
    x4 = _tc3_call(a3, _stack_w(W3, 256), invd, b3.reshape(1, 256))

    a4 = _scl_call(3, 4, 90, 70)(*x4, src2d, dst2d, gw, zeros)
    return _tc4_call(a4, _stack_w(W4, 256), invd, b4.reshape(1, 256))
